# baseline (device time: 170148 ns/iter reference)
import jax
import jax.numpy as jnp
from jax import lax
from jax.experimental import pallas as pl
from jax.experimental.pallas import tpu as pltpu

N_Z = 4
B, S_LOC, H, D = 4, 256, 16, 64
BH = B * H
QTR = BH // 4
SCALE = D ** -0.5

ZR, ZL = 0, 1
SX, SY, SF = 0, 1, 2


def kernel(Q, K, V):
    Qt = Q.transpose(0, 2, 1, 3).reshape(BH, S_LOC, D).astype(jnp.bfloat16)
    Kt = K.transpose(0, 2, 3, 1).reshape(BH, D, S_LOC).astype(jnp.bfloat16)
    Vt = V.transpose(0, 2, 3, 1).reshape(BH, D, S_LOC).astype(jnp.bfloat16)
    KVt = jnp.stack([Kt, Vt], axis=1)

    def body(q_ref, kv_ref, out_ref, kv_all, acc, send_z, recv_z,
             send_sq, recv_sq):
        my_x = lax.axis_index("x")
        my_y = lax.axis_index("y")
        my_z = lax.axis_index("z")
        x_nbr = (1 - my_x, my_y, my_z)
        y_nbr = (my_x, 1 - my_y, my_z)
        qid = my_x + 2 * my_y
        q_xn = (1 - my_x) + 2 * my_y
        q_yn = my_x + 2 * (1 - my_y)
        q_dg = (1 - my_x) + 2 * (1 - my_y)

        def blk(b):
            return pl.ds(QTR * b, QTR)

        ones_row = jnp.ones((1, S_LOC), jnp.bfloat16)

        kv_all[my_z] = kv_ref[...]

        barrier = pltpu.get_barrier_semaphore()
        for nbr in (x_nbr, y_nbr):
            pl.semaphore_signal(barrier, inc=1, device_id=nbr,
                                device_id_type=pl.DeviceIdType.MESH)
        for zi in range(N_Z):
            @pl.when(zi != my_z)
            def _(zi=zi):
                pl.semaphore_signal(barrier, inc=1,
                                    device_id=(my_x, my_y, zi),
                                    device_id_type=pl.DeviceIdType.MESH)
        pl.semaphore_wait(barrier, N_Z - 1 + 2)

        def zcopy(origin, direction, d, target_z):
            return pltpu.make_async_remote_copy(
                src_ref=kv_all.at[origin, blk(qid)],
                dst_ref=kv_all.at[origin, blk(qid)],
                send_sem=send_z.at[direction, d - 1],
                recv_sem=recv_z.at[direction, d - 1],
                device_id=(my_x, my_y, target_z),
                device_id_type=pl.DeviceIdType.MESH)

        def sqcopy(origin, b, d, side, slot, target):
            return pltpu.make_async_remote_copy(
                src_ref=kv_all.at[origin, blk(b)],
                dst_ref=kv_all.at[origin, blk(b)],
                send_sem=send_sq.at[d - 1, side, slot],
                recv_sem=recv_sq.at[d - 1, side, slot],
                device_id=target, device_id_type=pl.DeviceIdType.MESH)

        for d in range(1, N_Z):
            @pl.when(my_z + d <= N_Z - 1)
            def _(d=d):
                zcopy(my_z, ZR, d, my_z + d).start()

            @pl.when(my_z - d >= 0)
            def _(d=d):
                zcopy(my_z, ZL, d, my_z - d).start()

        def flash(origin, first):
            def bh_body(bh, c):
                q = q_ref[bh]
                kT = kv_all[origin, bh, 0]
                s = lax.dot_general(q, kT, (((1,), (0,)), ((), ())),
                                    preferred_element_type=jnp.float32)
                p = jnp.exp(s * SCALE).astype(jnp.bfloat16)
                vT = kv_all[origin, bh, 1]
                v_aug = jnp.concatenate([vT, ones_row], axis=0)
                contrib = lax.dot_general(
                    v_aug, p, (((1,), (1,)), ((), ())),
                    preferred_element_type=jnp.float32)
                if first:
                    acc[bh] = contrib
                else:
                    acc[bh] = acc[bh] + contrib
                return c
            lax.fori_loop(0, BH, bh_body, 0)

        flash(my_z, first=True)

        def origin_of(d, side):
            return my_z - d if side == 0 else my_z + d

        def valid(d, side):
            return (my_z - d >= 0) if side == 0 else (my_z + d <= N_Z - 1)

        for d in range(1, N_Z):
            for side in (0, 1):
                @pl.when(valid(d, side))
                def _(d=d, side=side):
                    o = origin_of(d, side)
                    zcopy(o, ZR if side == 0 else ZL, d, my_z).wait_recv()
                    sqcopy(o, qid, d, side, SX, x_nbr).start()
                    sqcopy(o, qid, d, side, SY, y_nbr).start()

        def fwd(d, side):
            @pl.when(valid(d, side))
            def _():
                o = origin_of(d, side)
                if (d + side) % 2 == 0:
                    sqcopy(o, q_xn, d, side, SX, x_nbr).wait_recv()
                    sqcopy(o, q_xn, d, side, SF, y_nbr).start()
                else:
                    sqcopy(o, q_yn, d, side, SY, y_nbr).wait_recv()
                    sqcopy(o, q_yn, d, side, SF, x_nbr).start()

        def compute_slot(d, side):
            @pl.when(valid(d, side))
            def _():
                o = origin_of(d, side)
                if (d + side) % 2 == 0:
                    sqcopy(o, q_yn, d, side, SY, y_nbr).wait_recv()
                    sqcopy(o, q_dg, d, side, SF, y_nbr).wait_recv()
                else:
                    sqcopy(o, q_xn, d, side, SX, x_nbr).wait_recv()
                    sqcopy(o, q_dg, d, side, SF, x_nbr).wait_recv()
                flash(o, first=False)

        fwd(1, 0)
        fwd(1, 1)
        compute_slot(1, 0)
        fwd(2, 0)
        fwd(2, 1)
        compute_slot(1, 1)
        fwd(3, 0)
        fwd(3, 1)
        compute_slot(2, 0)
        compute_slot(2, 1)
        compute_slot(3, 0)
        compute_slot(3, 1)

        def norm_body(bh, c):
            a = acc[bh]
            out_ref[bh] = a[0:D, :] / a[D:D + 1, :]
            return c
        lax.fori_loop(0, BH, norm_body, 0)

        for d in range(1, N_Z):
            @pl.when(my_z + d <= N_Z - 1)
            def _(d=d):
                zcopy(my_z, ZR, d, my_z + d).wait_send()

            @pl.when(my_z - d >= 0)
            def _(d=d):
                zcopy(my_z, ZL, d, my_z - d).wait_send()

            for side in (0, 1):
                @pl.when(valid(d, side))
                def _(d=d, side=side):
                    o = origin_of(d, side)
                    sqcopy(o, qid, d, side, SX, x_nbr).wait_send()
                    sqcopy(o, qid, d, side, SY, y_nbr).wait_send()
                    if (d + side) % 2 == 0:
                        sqcopy(o, q_xn, d, side, SF, y_nbr).wait_send()
                    else:
                        sqcopy(o, q_yn, d, side, SF, x_nbr).wait_send()

    out = pl.pallas_call(
        body,
        out_shape=jax.ShapeDtypeStruct((BH, D, S_LOC), jnp.float32),
        in_specs=[pl.BlockSpec(memory_space=pltpu.VMEM)] * 2,
        out_specs=pl.BlockSpec(memory_space=pltpu.VMEM),
        scratch_shapes=[
            pltpu.VMEM((N_Z, BH, 2, D, S_LOC), jnp.bfloat16),
            pltpu.VMEM((BH, D + 1, S_LOC), jnp.float32),
            pltpu.SemaphoreType.DMA((2, N_Z - 1)),
            pltpu.SemaphoreType.DMA((2, N_Z - 1)),
            pltpu.SemaphoreType.DMA((N_Z - 1, 2, 3)),
            pltpu.SemaphoreType.DMA((N_Z - 1, 2, 3)),
        ],
        compiler_params=pltpu.CompilerParams(
            collective_id=0, vmem_limit_bytes=48 * 1024 * 1024),
    )(Qt, KVt)

    return out.reshape(B, H, D, S_LOC).transpose(0, 3, 1, 2)


# device time: 132015 ns/iter; 1.2889x vs baseline; 1.2889x over previous
import jax
import jax.numpy as jnp
from jax import lax
from jax.experimental import pallas as pl
from jax.experimental.pallas import tpu as pltpu

N_Z = 4
B, S_LOC, H, D = 4, 256, 16, 64
BH = B * H
QTR = BH // 4
CG = 8
SCALE = D ** -0.5

ZR, ZL = 0, 1
SX, SY, SF = 0, 1, 2


def kernel(Q, K, V):
    Qt = Q.transpose(0, 2, 1, 3).reshape(BH, S_LOC, D).astype(jnp.bfloat16)
    Kt = K.transpose(0, 2, 3, 1).reshape(BH, D, S_LOC).astype(jnp.bfloat16)
    Vt = V.transpose(0, 2, 3, 1).reshape(BH, D, S_LOC).astype(jnp.bfloat16)
    KVt = jnp.stack([Kt, Vt], axis=1)

    def body(q_ref, kv_ref, out_ref, kv_all, acc, send_z, recv_z,
             send_sq, recv_sq):
        my_x = lax.axis_index("x")
        my_y = lax.axis_index("y")
        my_z = lax.axis_index("z")
        x_nbr = (1 - my_x, my_y, my_z)
        y_nbr = (my_x, 1 - my_y, my_z)
        qid = my_x + 2 * my_y
        q_xn = (1 - my_x) + 2 * my_y
        q_yn = my_x + 2 * (1 - my_y)
        q_dg = (1 - my_x) + 2 * (1 - my_y)

        def blk(b):
            return pl.ds(QTR * b, QTR)

        ones_l = jnp.ones((CG, 1, S_LOC), jnp.bfloat16)

        kv_all[my_z] = kv_ref[...]

        barrier = pltpu.get_barrier_semaphore()
        for nbr in (x_nbr, y_nbr):
            pl.semaphore_signal(barrier, inc=1, device_id=nbr,
                                device_id_type=pl.DeviceIdType.MESH)
        for zi in range(N_Z):
            @pl.when(zi != my_z)
            def _(zi=zi):
                pl.semaphore_signal(barrier, inc=1,
                                    device_id=(my_x, my_y, zi),
                                    device_id_type=pl.DeviceIdType.MESH)
        pl.semaphore_wait(barrier, N_Z - 1 + 2)

        def zcopy(origin, direction, d, target_z):
            return pltpu.make_async_remote_copy(
                src_ref=kv_all.at[origin, blk(qid)],
                dst_ref=kv_all.at[origin, blk(qid)],
                send_sem=send_z.at[direction, d - 1],
                recv_sem=recv_z.at[direction, d - 1],
                device_id=(my_x, my_y, target_z),
                device_id_type=pl.DeviceIdType.MESH)

        def sqcopy(origin, b, d, side, slot, target):
            return pltpu.make_async_remote_copy(
                src_ref=kv_all.at[origin, blk(b)],
                dst_ref=kv_all.at[origin, blk(b)],
                send_sem=send_sq.at[d - 1, side, slot],
                recv_sem=recv_sq.at[d - 1, side, slot],
                device_id=target, device_id_type=pl.DeviceIdType.MESH)

        for d in range(1, N_Z):
            @pl.when(my_z + d <= N_Z - 1)
            def _(d=d):
                zcopy(my_z, ZR, d, my_z + d).start()

            @pl.when(my_z - d >= 0)
            def _(d=d):
                zcopy(my_z, ZL, d, my_z - d).start()

        def flash(origin, first):
            def g_body(g, c):
                r = pl.ds(g * CG, CG)
                q = q_ref[r]
                kT = kv_all[origin, r, 0]
                s = lax.dot_general(
                    q, kT, (((2,), (1,)), ((0,), (0,))),
                    preferred_element_type=jnp.float32)
                p = jnp.exp(s * SCALE).astype(jnp.bfloat16)
                vT = kv_all[origin, r, 1]
                pv = lax.dot_general(
                    vT, p, (((2,), (2,)), ((0,), (0,))),
                    preferred_element_type=jnp.float32)
                lrow = lax.dot_general(
                    ones_l, p, (((2,), (2,)), ((0,), (0,))),
                    preferred_element_type=jnp.float32)
                if first:
                    acc[r, 0:D] = pv
                    acc[r, D:D + 1] = lrow
                else:
                    acc[r, 0:D] = acc[r, 0:D] + pv
                    acc[r, D:D + 1] = acc[r, D:D + 1] + lrow
                return c
            lax.fori_loop(0, BH // CG, g_body, 0)

        flash(my_z, first=True)

        def origin_of(d, side):
            return my_z - d if side == 0 else my_z + d

        def valid(d, side):
            return (my_z - d >= 0) if side == 0 else (my_z + d <= N_Z - 1)

        for d in range(1, N_Z):
            for side in (0, 1):
                @pl.when(valid(d, side))
                def _(d=d, side=side):
                    o = origin_of(d, side)
                    zcopy(o, ZR if side == 0 else ZL, d, my_z).wait_recv()
                    sqcopy(o, qid, d, side, SX, x_nbr).start()
                    sqcopy(o, qid, d, side, SY, y_nbr).start()

        def fwd(d, side):
            @pl.when(valid(d, side))
            def _():
                o = origin_of(d, side)
                if (d + side) % 2 == 0:
                    sqcopy(o, q_xn, d, side, SX, x_nbr).wait_recv()
                    sqcopy(o, q_xn, d, side, SF, y_nbr).start()
                else:
                    sqcopy(o, q_yn, d, side, SY, y_nbr).wait_recv()
                    sqcopy(o, q_yn, d, side, SF, x_nbr).start()

        def compute_slot(d, side):
            @pl.when(valid(d, side))
            def _():
                o = origin_of(d, side)
                if (d + side) % 2 == 0:
                    sqcopy(o, q_yn, d, side, SY, y_nbr).wait_recv()
                    sqcopy(o, q_dg, d, side, SF, y_nbr).wait_recv()
                else:
                    sqcopy(o, q_xn, d, side, SX, x_nbr).wait_recv()
                    sqcopy(o, q_dg, d, side, SF, x_nbr).wait_recv()
                flash(o, first=False)

        fwd(1, 0)
        fwd(1, 1)
        compute_slot(1, 0)
        fwd(2, 0)
        fwd(2, 1)
        compute_slot(1, 1)
        fwd(3, 0)
        fwd(3, 1)
        compute_slot(2, 0)
        compute_slot(2, 1)
        compute_slot(3, 0)
        compute_slot(3, 1)

        def norm_body(g, c):
            r = pl.ds(g * CG, CG)
            out_ref[r] = acc[r, 0:D] / acc[r, D:D + 1]
            return c
        lax.fori_loop(0, BH // CG, norm_body, 0)

        for d in range(1, N_Z):
            @pl.when(my_z + d <= N_Z - 1)
            def _(d=d):
                zcopy(my_z, ZR, d, my_z + d).wait_send()

            @pl.when(my_z - d >= 0)
            def _(d=d):
                zcopy(my_z, ZL, d, my_z - d).wait_send()

            for side in (0, 1):
                @pl.when(valid(d, side))
                def _(d=d, side=side):
                    o = origin_of(d, side)
                    sqcopy(o, qid, d, side, SX, x_nbr).wait_send()
                    sqcopy(o, qid, d, side, SY, y_nbr).wait_send()
                    if (d + side) % 2 == 0:
                        sqcopy(o, q_xn, d, side, SF, y_nbr).wait_send()
                    else:
                        sqcopy(o, q_yn, d, side, SF, x_nbr).wait_send()

    out = pl.pallas_call(
        body,
        out_shape=jax.ShapeDtypeStruct((BH, D, S_LOC), jnp.float32),
        in_specs=[pl.BlockSpec(memory_space=pltpu.VMEM)] * 2,
        out_specs=pl.BlockSpec(memory_space=pltpu.VMEM),
        scratch_shapes=[
            pltpu.VMEM((N_Z, BH, 2, D, S_LOC), jnp.bfloat16),
            pltpu.VMEM((BH, D + 1, S_LOC), jnp.float32),
            pltpu.SemaphoreType.DMA((2, N_Z - 1)),
            pltpu.SemaphoreType.DMA((2, N_Z - 1)),
            pltpu.SemaphoreType.DMA((N_Z - 1, 2, 3)),
            pltpu.SemaphoreType.DMA((N_Z - 1, 2, 3)),
        ],
        compiler_params=pltpu.CompilerParams(
            collective_id=0, vmem_limit_bytes=48 * 1024 * 1024),
    )(Qt, KVt)

    return out.reshape(B, H, D, S_LOC).transpose(0, 3, 1, 2)


# device time: 112528 ns/iter; 1.5121x vs baseline; 1.1732x over previous
import jax
import jax.numpy as jnp
from jax import lax
from jax.experimental import pallas as pl
from jax.experimental.pallas import tpu as pltpu

N_Z = 4
B, S_LOC, H, D = 4, 256, 16, 64
BH = B * H
QTR = BH // 4
SCALE = D ** -0.5


def kernel(Q, K, V):
    Qt = Q.transpose(0, 2, 1, 3).reshape(BH, S_LOC, D).astype(jnp.bfloat16)
    Kt = K.transpose(0, 2, 3, 1).reshape(BH, D, S_LOC).astype(jnp.bfloat16)
    Vt = V.transpose(0, 2, 3, 1).reshape(BH, D, S_LOC).astype(jnp.bfloat16)
    KVt = jnp.stack([Kt, Vt], axis=1)

    def body(q_ref, kv_ref, out_ref, qg, part_own, pbuf_out, pbuf_in,
             cbuf_out, cbuf_in, qsend, qrecv, psend, precv, csend, crecv):
        my_x = lax.axis_index("x")
        my_y = lax.axis_index("y")
        my_z = lax.axis_index("z")
        qid = my_x + 2 * my_y
        qblk = pl.ds(QTR * qid, QTR)

        ones_l = jnp.ones((QTR, 1, S_LOC), jnp.bfloat16)

        barrier = pltpu.get_barrier_semaphore()
        for zi in range(N_Z):
            @pl.when(zi != my_z)
            def _(zi=zi):
                pl.semaphore_signal(barrier, inc=1,
                                    device_id=(my_x, my_y, zi),
                                    device_id_type=pl.DeviceIdType.MESH)
        for p in range(4):
            @pl.when(p != qid)
            def _(p=p):
                pl.semaphore_signal(barrier, inc=1,
                                    device_id=(p % 2, p // 2, my_z),
                                    device_id_type=pl.DeviceIdType.MESH)
        pl.semaphore_wait(barrier, 6)

        for zi in range(N_Z):
            @pl.when(zi != my_z)
            def _(zi=zi):
                pltpu.make_async_remote_copy(
                    src_ref=q_ref.at[qblk], dst_ref=qg.at[my_z],
                    send_sem=qsend.at[zi], recv_sem=qrecv.at[my_z],
                    device_id=(my_x, my_y, zi),
                    device_id_type=pl.DeviceIdType.MESH).start()

        def partial(qq):
            kT = kv_ref[qblk, 0]
            s = lax.dot_general(
                qq, kT, (((2,), (1,)), ((0,), (0,))),
                preferred_element_type=jnp.float32)
            p = jnp.exp(s * SCALE).astype(jnp.bfloat16)
            vT = kv_ref[qblk, 1]
            pv = lax.dot_general(
                vT, p, (((2,), (2,)), ((0,), (0,))),
                preferred_element_type=jnp.float32)
            lrow = lax.dot_general(
                ones_l, p, (((2,), (2,)), ((0,), (0,))),
                preferred_element_type=jnp.float32)
            return jnp.concatenate([pv, lrow], axis=1)

        part_own[...] = partial(q_ref[qblk])

        for d in range(1, N_Z):
            for side in (0, 1):
                @pl.when((my_z - d >= 0) if side == 0
                         else (my_z + d <= N_Z - 1))
                def _(d=d, side=side):
                    o = my_z - d if side == 0 else my_z + d
                    pltpu.make_async_remote_copy(
                        src_ref=q_ref.at[qblk], dst_ref=qg.at[o],
                        send_sem=qsend.at[o], recv_sem=qrecv.at[o],
                        device_id=(my_x, my_y, o),
                        device_id_type=pl.DeviceIdType.MESH).wait_recv()
                    pbuf_out[o] = partial(qg[o]).astype(jnp.bfloat16)
                    pltpu.make_async_remote_copy(
                        src_ref=pbuf_out.at[o], dst_ref=pbuf_in.at[my_z],
                        send_sem=psend.at[o], recv_sem=precv.at[my_z],
                        device_id=(my_x, my_y, o),
                        device_id_type=pl.DeviceIdType.MESH).start()

        for c in range(N_Z):
            @pl.when(c != my_z)
            def _(c=c):
                pltpu.make_async_remote_copy(
                    src_ref=pbuf_out.at[c], dst_ref=pbuf_in.at[c],
                    send_sem=psend.at[c], recv_sem=precv.at[c],
                    device_id=(my_x, my_y, c),
                    device_id_type=pl.DeviceIdType.MESH).wait_recv()

        comb = part_own[...]
        for c in range(N_Z):
            comb = comb + jnp.where(c == my_z, 0.0,
                                    pbuf_in[c].astype(jnp.float32))
        cbuf_out[...] = comb.astype(jnp.bfloat16)

        out_ref[qblk] = comb[:, 0:D] / comb[:, D:D + 1]

        for p in range(4):
            @pl.when(p != qid)
            def _(p=p):
                pltpu.make_async_remote_copy(
                    src_ref=cbuf_out, dst_ref=cbuf_in.at[qid],
                    send_sem=csend.at[p], recv_sem=crecv.at[qid],
                    device_id=(p % 2, p // 2, my_z),
                    device_id_type=pl.DeviceIdType.MESH).start()

        for p in range(4):
            @pl.when(p != qid)
            def _(p=p):
                pltpu.make_async_remote_copy(
                    src_ref=cbuf_out, dst_ref=cbuf_in.at[p],
                    send_sem=csend.at[p], recv_sem=crecv.at[p],
                    device_id=(p % 2, p // 2, my_z),
                    device_id_type=pl.DeviceIdType.MESH).wait_recv()
                a = cbuf_in[p].astype(jnp.float32)
                out_ref[QTR * p:QTR * (p + 1)] = a[:, 0:D] / a[:, D:D + 1]

        for zi in range(N_Z):
            @pl.when(zi != my_z)
            def _(zi=zi):
                pltpu.make_async_remote_copy(
                    src_ref=q_ref.at[qblk], dst_ref=qg.at[my_z],
                    send_sem=qsend.at[zi], recv_sem=qrecv.at[my_z],
                    device_id=(my_x, my_y, zi),
                    device_id_type=pl.DeviceIdType.MESH).wait_send()

                pltpu.make_async_remote_copy(
                    src_ref=pbuf_out.at[zi], dst_ref=pbuf_in.at[my_z],
                    send_sem=psend.at[zi], recv_sem=precv.at[my_z],
                    device_id=(my_x, my_y, zi),
                    device_id_type=pl.DeviceIdType.MESH).wait_send()
        for p in range(4):
            @pl.when(p != qid)
            def _(p=p):
                pltpu.make_async_remote_copy(
                    src_ref=cbuf_out, dst_ref=cbuf_in.at[qid],
                    send_sem=csend.at[p], recv_sem=crecv.at[qid],
                    device_id=(p % 2, p // 2, my_z),
                    device_id_type=pl.DeviceIdType.MESH).wait_send()

    out = pl.pallas_call(
        body,
        out_shape=jax.ShapeDtypeStruct((BH, D, S_LOC), jnp.float32),
        in_specs=[pl.BlockSpec(memory_space=pltpu.VMEM)] * 2,
        out_specs=pl.BlockSpec(memory_space=pltpu.VMEM),
        scratch_shapes=[
            pltpu.VMEM((N_Z, QTR, S_LOC, D), jnp.bfloat16),
            pltpu.VMEM((QTR, D + 1, S_LOC), jnp.float32),
            pltpu.VMEM((N_Z, QTR, D + 1, S_LOC), jnp.bfloat16),
            pltpu.VMEM((N_Z, QTR, D + 1, S_LOC), jnp.bfloat16),
            pltpu.VMEM((QTR, D + 1, S_LOC), jnp.bfloat16),
            pltpu.VMEM((4, QTR, D + 1, S_LOC), jnp.bfloat16),
            pltpu.SemaphoreType.DMA((N_Z,)),
            pltpu.SemaphoreType.DMA((N_Z,)),
            pltpu.SemaphoreType.DMA((N_Z,)),
            pltpu.SemaphoreType.DMA((N_Z,)),
            pltpu.SemaphoreType.DMA((4,)),
            pltpu.SemaphoreType.DMA((4,)),
        ],
        compiler_params=pltpu.CompilerParams(
            collective_id=0, vmem_limit_bytes=48 * 1024 * 1024),
    )(Qt, KVt)

    return out.reshape(B, H, D, S_LOC).transpose(0, 3, 1, 2)
